# Initial kernel scaffold; baseline (speedup 1.0000x reference)
import jax
import jax.numpy as jnp
from jax.experimental import pallas as pl


def kernel(input, edge_index, edge_weight, W1, b1, W2, b2):
    raise NotImplementedError("write your pallas kernel here")


# speedup vs baseline: 8.5964x; 8.5964x over previous
import functools

import jax
import jax.numpy as jnp
from jax import lax
from jax.experimental import pallas as pl
from jax.experimental.pallas import tpu as pltpu
from jax.experimental.pallas import tpu_sc as plsc

N = 10000
D = 256
H = 128
E = 160000
NC = 2
NS = 16
NPAD = 10240
K = 128
NCH_DEG = 40
EPT_DEG = NCH_DEG * K
E_PAD = NC * NS * EPT_DEG
NCH = 79
EPT = NCH * K
E_PAD_ACC = NS * EPT
R = 1000

_mesh = plsc.VectorSubcoreMesh(core_axis_name="c", subcore_axis_name="s")


@functools.partial(
    pl.kernel,
    out_type=jax.ShapeDtypeStruct((NC * NPAD,), jnp.float32),
    mesh=_mesh,
    scratch_types=[
        pltpu.VMEM((NCH_DEG, K), jnp.int32),
        pltpu.VMEM((NCH_DEG, K), jnp.float32),
        pltpu.VMEM((640,), jnp.float32),
        pltpu.VMEM_SHARED((NPAD,), jnp.float32),
    ],
)
def _sc_deg(dst_hbm, ew_hbm, deg_out, dst_v, ew_v, zbuf, deg_sh):
    cid = lax.axis_index("c")
    sid = lax.axis_index("s")
    wid = cid * NS + sid
    zero16 = jnp.zeros((16,), jnp.float32)

    pltpu.sync_copy(dst_hbm.at[wid], dst_v)
    pltpu.sync_copy(ew_hbm.at[wid], ew_v)

    def zf(i, _):
        zbuf[pl.ds(i * 16, 16)] = zero16
        return 0
    lax.fori_loop(0, 640 // 16, zf, 0)
    rpt = NPAD // NS
    pltpu.sync_copy(zbuf, deg_sh.at[pl.ds(sid * rpt, rpt)])
    plsc.subcore_barrier()

    def chunk_body(k, _):
        pltpu.sync_copy(ew_v.at[k], deg_sh.at[dst_v.at[k]], add=True)
        return 0
    lax.fori_loop(0, NCH_DEG, chunk_body, 0)
    plsc.subcore_barrier()

    pltpu.sync_copy(deg_sh.at[pl.ds(sid * rpt, rpt)],
                    deg_out.at[pl.ds(cid * NPAD + sid * rpt, rpt)])


@functools.partial(
    pl.kernel,
    out_type=[jax.ShapeDtypeStruct((N, H), jnp.float32),
              jax.ShapeDtypeStruct((N, H), jnp.float32)],
    mesh=_mesh,
    scratch_types=[
        pltpu.VMEM((NCH, K), jnp.int32),
        pltpu.VMEM((NCH, K), jnp.int32),
        pltpu.VMEM((NCH, K), jnp.float32),
        pltpu.VMEM((K, H), jnp.float32),
        pltpu.VMEM_SHARED((N, H), jnp.float32),
        pltpu.SemaphoreType.DMA,
    ],
)
def _sc_acc(y0_hbm, y1_hbm, src_hbm, dst_hbm, ew_hbm,
            out0_hbm, out1_hbm, src_v, dst_v, ew_v, gbuf, acc_sh, sem):
    cid = lax.axis_index("c")
    sid = lax.axis_index("s")

    pltpu.sync_copy(src_hbm.at[sid], src_v)
    pltpu.sync_copy(dst_hbm.at[sid], dst_v)
    pltpu.sync_copy(ew_hbm.at[sid], ew_v)

    zero16 = jnp.zeros((16,), jnp.float32)

    def zg(i, _):
        for j in range(H // 16):
            gbuf[i, pl.ds(j * 16, 16)] = zero16
        return 0
    lax.fori_loop(0, K, zg, 0)
    rpt = N // NS
    for cpy in range(4):
        pltpu.sync_copy(gbuf.at[pl.ds(0, K)],
                        acc_sh.at[pl.ds(sid * rpt + cpy * K, K)])
    pltpu.sync_copy(gbuf.at[pl.ds(0, rpt - 4 * K)],
                    acc_sh.at[pl.ds(sid * rpt + 4 * K, rpt - 4 * K)])
    plsc.subcore_barrier()

    def chunk_body(k, _):
        idx = src_v.at[k]

        @pl.when(cid == 0)
        def _():
            pltpu.async_copy(y0_hbm.at[idx], gbuf, sem)

        @pl.when(cid == 1)
        def _():
            pltpu.async_copy(y1_hbm.at[idx], gbuf, sem)

        pltpu.make_async_copy(y0_hbm.at[idx], gbuf, sem).wait()

        def grp_body(g, _):
            wvec = ew_v[k, pl.ds(g * 16, 16)]
            for j in range(16):
                w = wvec[j]
                e = g * 16 + j
                for f in range(H // 16):
                    sl = pl.ds(f * 16, 16)
                    gbuf[e, sl] = gbuf[e, sl] * w
            return 0
        lax.fori_loop(0, K // 16, grp_body, 0)

        pltpu.sync_copy(gbuf, acc_sh.at[dst_v.at[k]], add=True)
        return 0
    lax.fori_loop(0, NCH, chunk_body, 0)
    plsc.subcore_barrier()

    row8 = sid * 624

    @pl.when(cid == 0)
    def _():
        pltpu.sync_copy(acc_sh.at[pl.ds(row8, 624)], out0_hbm.at[pl.ds(row8, 624)])

    @pl.when(cid == 1)
    def _():
        pltpu.sync_copy(acc_sh.at[pl.ds(row8, 624)], out1_hbm.at[pl.ds(row8, 624)])

    @pl.when((cid == 0) & (sid == NS - 1))
    def _():
        pltpu.sync_copy(acc_sh.at[pl.ds(9984, 16)], out0_hbm.at[pl.ds(9984, 16)])

    @pl.when((cid == 1) & (sid == NS - 1))
    def _():
        pltpu.sync_copy(acc_sh.at[pl.ds(9984, 16)], out1_hbm.at[pl.ds(9984, 16)])


def _dis_of(deg_ref):
    return lax.rsqrt(deg_ref[...])


def _mm1_body(x_ref, w_ref, deg_ref, y0_ref, y1_ref):
    dis = _dis_of(deg_ref)
    xw = jnp.dot(x_ref[...], w_ref[...], preferred_element_type=jnp.float32)
    y = dis * xw
    y0_ref[...] = y[:, :H]
    y1_ref[...] = y[:, H:]


_DEG_SPEC = pl.BlockSpec((R, 1), lambda i: (i, 0))


def _tc_mm1(x, w1, deg_parts):
    return pl.pallas_call(
        _mm1_body,
        grid=(N // R,),
        in_specs=[
            pl.BlockSpec((R, D), lambda i: (i, 0)),
            pl.BlockSpec((D, D), lambda i: (0, 0)),
            _DEG_SPEC,
        ],
        out_specs=[
            pl.BlockSpec((R, H), lambda i: (i, 0)),
            pl.BlockSpec((R, H), lambda i: (i, 0)),
        ],
        out_shape=[jax.ShapeDtypeStruct((N, H), jnp.float32),
                   jax.ShapeDtypeStruct((N, H), jnp.float32)],
    )(x, w1, deg_parts)


def _mm2_body(a0_ref, a1_ref, y0_ref, y1_ref, deg_ref, b1_ref, w2_ref,
              z0_ref, z1_ref):
    dis = _dis_of(deg_ref)
    b1 = b1_ref[...]
    h0 = jnp.maximum(dis * (a0_ref[...] + y0_ref[...]) + b1[:H][None, :], 0.0)
    h1 = jnp.maximum(dis * (a1_ref[...] + y1_ref[...]) + b1[H:][None, :], 0.0)
    h = jnp.concatenate([h0, h1], axis=1)
    xw = jnp.dot(h, w2_ref[...], preferred_element_type=jnp.float32)
    z = dis * xw
    z0_ref[...] = z[:, :H]
    z1_ref[...] = z[:, H:]


def _tc_mm2(a0, a1, y0, y1, deg_parts, b1, w2):
    return pl.pallas_call(
        _mm2_body,
        grid=(N // R,),
        in_specs=[
            pl.BlockSpec((R, H), lambda i: (i, 0)),
            pl.BlockSpec((R, H), lambda i: (i, 0)),
            pl.BlockSpec((R, H), lambda i: (i, 0)),
            pl.BlockSpec((R, H), lambda i: (i, 0)),
            _DEG_SPEC,
            pl.BlockSpec((D,), lambda i: (0,)),
            pl.BlockSpec((D, D), lambda i: (0, 0)),
        ],
        out_specs=[
            pl.BlockSpec((R, H), lambda i: (i, 0)),
            pl.BlockSpec((R, H), lambda i: (i, 0)),
        ],
        out_shape=[jax.ShapeDtypeStruct((N, H), jnp.float32),
                   jax.ShapeDtypeStruct((N, H), jnp.float32)],
    )(a0, a1, y0, y1, deg_parts, b1, w2)


def _out_body(a0_ref, a1_ref, y0_ref, y1_ref, deg_ref, b2_ref, o_ref):
    dis = _dis_of(deg_ref)
    b2 = b2_ref[...]
    o0 = dis * (a0_ref[...] + y0_ref[...]) + b2[:H][None, :]
    o1 = dis * (a1_ref[...] + y1_ref[...]) + b2[H:][None, :]
    o = jnp.concatenate([o0, o1], axis=1)
    m = jnp.max(o, axis=1, keepdims=True)
    e = jnp.exp(o - m)
    o_ref[...] = e / jnp.sum(e, axis=1, keepdims=True)


def _tc_out(a0, a1, y0, y1, deg_parts, b2):
    return pl.pallas_call(
        _out_body,
        grid=(N // R,),
        in_specs=[
            pl.BlockSpec((R, H), lambda i: (i, 0)),
            pl.BlockSpec((R, H), lambda i: (i, 0)),
            pl.BlockSpec((R, H), lambda i: (i, 0)),
            pl.BlockSpec((R, H), lambda i: (i, 0)),
            _DEG_SPEC,
            pl.BlockSpec((D,), lambda i: (0,)),
        ],
        out_specs=pl.BlockSpec((R, D), lambda i: (i, 0)),
        out_shape=jax.ShapeDtypeStruct((N, D), jnp.float32),
    )(a0, a1, y0, y1, deg_parts, b2)


def kernel(input, edge_index, edge_weight, W1, b1, W2, b2):
    src = edge_index[0].astype(jnp.int32)
    dst = edge_index[1].astype(jnp.int32)
    ew = edge_weight.astype(jnp.float32)

    pad = E_PAD - E
    dst_p = jnp.concatenate([dst, jnp.zeros((pad,), jnp.int32)])
    ew_p = jnp.concatenate([ew, jnp.zeros((pad,), jnp.float32)])
    deg_parts = _sc_deg(dst_p.reshape(NC * NS, NCH_DEG, K),
                        ew_p.reshape(NC * NS, NCH_DEG, K))
    deg_col = (deg_parts[:NPAD] + deg_parts[NPAD:] + 1.0)[:N, None]

    pad_a = E_PAD_ACC - E
    src_t = jnp.concatenate([src, jnp.zeros((pad_a,), jnp.int32)]).reshape(NS, NCH, K)
    dst_t = jnp.concatenate([dst, jnp.zeros((pad_a,), jnp.int32)]).reshape(NS, NCH, K)
    ew_t = jnp.concatenate([ew, jnp.zeros((pad_a,), jnp.float32)]).reshape(NS, NCH, K)

    y0, y1 = _tc_mm1(input, W1, deg_col)
    a0, a1 = _sc_acc(y0, y1, src_t, dst_t, ew_t)
    z0, z1 = _tc_mm2(a0, a1, y0, y1, deg_col, b1, W2)
    c0, c1 = _sc_acc(z0, z1, src_t, dst_t, ew_t)
    return _tc_out(c0, c1, z0, z1, deg_col, b2)
